# baseline (speedup 1.0000x reference)
import functools

import jax
import jax.numpy as jnp
from jax import lax
from jax.experimental import pallas as pl
from jax.experimental.pallas import tpu as pltpu
from jax.experimental.pallas import tpu_sc as plsc

N_CTRL = 32
NUM_CORES = 2
NUM_SUBCORES = 16


@functools.cache
def _make_kernel(B, D):
    num_cores = 1
    n_workers = num_cores * NUM_SUBCORES
    rows_total = B * N_CTRL
    rows_per_w = rows_total // n_workers
    mesh = plsc.VectorSubcoreMesh(core_axis_name="c", subcore_axis_name="s",
                                  num_cores=num_cores)

    half = rows_per_w // 2

    @functools.partial(
        pl.kernel,
        mesh=mesh,
        out_type=jax.ShapeDtypeStruct((rows_total, D), jnp.float32),
        scratch_types=[
            pltpu.VMEM((rows_per_w, D), jnp.float32),
            pltpu.SemaphoreType.DMA,
            pltpu.SemaphoreType.DMA,
        ],
    )
    def tile_copy(table_hbm, out_hbm, buf, sem_a, sem_b):
        wid = lax.axis_index("s") * num_cores + lax.axis_index("c")
        out_base = wid * rows_per_w
        tab_base = out_base % N_CTRL
        g0 = pltpu.async_copy(table_hbm.at[pl.ds(tab_base, half), :],
                              buf.at[pl.ds(0, half), :], sem_a)
        g1 = pltpu.async_copy(table_hbm.at[pl.ds(tab_base + half, half), :],
                              buf.at[pl.ds(half, half), :], sem_b)
        g0.wait()
        s0 = pltpu.async_copy(buf.at[pl.ds(0, half), :],
                              out_hbm.at[pl.ds(out_base, half), :], sem_a)
        g1.wait()
        s1 = pltpu.async_copy(buf.at[pl.ds(half, half), :],
                              out_hbm.at[pl.ds(out_base + half, half), :],
                              sem_b)
        s0.wait()
        s1.wait()

    return tile_copy


def kernel(x, embed_table):
    B = x.shape[0]
    D = embed_table.shape[1]
    out_flat = _make_kernel(B, D)(embed_table)
    return out_flat.reshape(B, N_CTRL, D)


# speedup vs baseline: 3.3343x; 3.3343x over previous
import functools

import jax
import jax.numpy as jnp
from jax.experimental import pallas as pl

N_CTRL = 32


def _body(table_ref, out_ref):
    out_ref[...] = table_ref[...][None]


@functools.cache
def _make_kernel(B, D):
    return pl.pallas_call(
        _body,
        grid=(B,),
        in_specs=[pl.BlockSpec((N_CTRL, D), lambda b: (0, 0))],
        out_specs=pl.BlockSpec((1, N_CTRL, D), lambda b: (b, 0, 0)),
        out_shape=jax.ShapeDtypeStruct((B, N_CTRL, D), jnp.float32),
    )


def kernel(x, embed_table):
    B = x.shape[0]
    D = embed_table.shape[1]
    return _make_kernel(B, D)(embed_table)


# speedup vs baseline: 11.4916x; 3.4465x over previous
import functools

import jax
import jax.numpy as jnp
from jax.experimental import pallas as pl

N_CTRL = 32


def _body(table_ref, out_ref):
    out_ref[...] = jnp.broadcast_to(table_ref[...][None], out_ref.shape)


@functools.cache
def _make_kernel(B, D):
    return pl.pallas_call(
        _body,
        grid=(1,),
        in_specs=[pl.BlockSpec((N_CTRL, D), lambda g: (0, 0))],
        out_specs=pl.BlockSpec((B, N_CTRL, D), lambda g: (0, 0, 0)),
        out_shape=jax.ShapeDtypeStruct((B, N_CTRL, D), jnp.float32),
    )


def kernel(x, embed_table):
    B = x.shape[0]
    D = embed_table.shape[1]
    return _make_kernel(B, D)(embed_table)
